# Initial kernel scaffold; baseline (speedup 1.0000x reference)
import jax
import jax.numpy as jnp
from jax.experimental import pallas as pl


def kernel(queries, keys, label_confi):
    raise NotImplementedError("write your pallas kernel here")


# speedup vs baseline: 1.1259x; 1.1259x over previous
import functools

import jax
import jax.numpy as jnp
from jax import lax
from jax.experimental import pallas as pl
from jax.experimental.pallas import tpu as pltpu
from jax.experimental.pallas import tpu_sc as plsc

Q = 1024
D = 128
KB = 2000


def _argmin_body(q_ref, k_ref, lab_ref, idx_ref, bestv_ref):
    j = pl.program_id(0)

    @pl.when(j == 0)
    def _init():
        idx_ref[...] = jnp.zeros_like(idx_ref)
        bestv_ref[...] = jnp.full_like(bestv_ref, jnp.inf)

    q = q_ref[...]
    qn = (q / (jnp.sqrt(jnp.sum(q * q, axis=1, keepdims=True)) + 1e-12)
          ).astype(jnp.bfloat16)
    k = k_ref[...]
    kn = (k / (jnp.sqrt(jnp.sum(k * k, axis=1, keepdims=True)) + 1e-12)
          ).astype(jnp.bfloat16)

    s = lax.dot_general(
        qn, kn,
        dimension_numbers=(((1,), (1,)), ((), ())),
        preferred_element_type=jnp.float32,
    )
    dd = 1.0 - s
    lab = lab_ref[...].reshape(1, KB)
    dd = jnp.where(lab > 0, dd, jnp.inf)

    local_min = jnp.min(dd, axis=1, keepdims=True)
    local_arg = jnp.argmin(dd, axis=1).astype(jnp.int32)
    local_arg = local_arg.reshape(Q, 1) + j * KB

    upd = local_min < bestv_ref[...]
    idx_ref[...] = jnp.where(upd, local_arg, idx_ref[...])
    bestv_ref[...] = jnp.where(upd, local_min, bestv_ref[...])


def _nearest_index(queries, keys, label_confi):
    K = keys.shape[0]
    nblk = K // KB
    lab3d = label_confi.reshape(K // KB, 1, KB).astype(jnp.int32)
    idx = pl.pallas_call(
        _argmin_body,
        grid=(nblk,),
        in_specs=[
            pl.BlockSpec((Q, D), lambda j: (0, 0)),
            pl.BlockSpec((KB, D), lambda j: (j, 0)),
            pl.BlockSpec((1, 1, KB), lambda j: (j, 0, 0)),
        ],
        out_specs=pl.BlockSpec((Q, 1), lambda j: (0, 0)),
        out_shape=jax.ShapeDtypeStruct((Q, 1), jnp.int32),
        scratch_shapes=[pltpu.VMEM((Q, 1), jnp.float32)],
    )(queries, keys, lab3d)
    return idx.reshape(Q)


def _make_sc_gather(V, B, Dm):
    NC, NS = 2, 16
    NW = NC * NS
    b_per_w = B // NW
    mesh = plsc.VectorSubcoreMesh(core_axis_name="c", subcore_axis_name="s")

    @functools.partial(
        pl.kernel,
        mesh=mesh,
        out_type=jax.ShapeDtypeStruct((B, Dm), jnp.float32),
        scratch_types=[
            pltpu.VMEM((b_per_w,), jnp.int32),
            pltpu.VMEM((b_per_w, Dm), jnp.float32),
            pltpu.SemaphoreType.DMA,
        ],
    )
    def gather_rows(idx_hbm, table_hbm, out_hbm, idx_v, rows_v, sem):
        wid = lax.axis_index("s") * NC + lax.axis_index("c")
        base = wid * b_per_w
        pltpu.sync_copy(idx_hbm.at[pl.ds(base, b_per_w)], idx_v)
        pltpu.async_copy(table_hbm.at[idx_v], rows_v, sem).wait()
        pltpu.sync_copy(rows_v, out_hbm.at[pl.ds(base, b_per_w)])

    return gather_rows


def kernel(queries, keys, label_confi):
    nearest_idx = _nearest_index(queries, keys, label_confi)
    gather = _make_sc_gather(keys.shape[0], Q, D)
    return gather(nearest_idx, keys)


# speedup vs baseline: 1.8979x; 1.6857x over previous
import functools

import jax
import jax.numpy as jnp
from jax import lax
from jax.experimental import pallas as pl
from jax.experimental.pallas import tpu as pltpu
from jax.experimental.pallas import tpu_sc as plsc

Q = 1024
D = 128
KB = 2000


def _argmin_body(q_ref, k_ref, lab_ref, idx_ref, qn_ref, rmin_ref, rjdx_ref):
    j = pl.program_id(0)
    nblk = pl.num_programs(0)

    @pl.when(j == 0)
    def _init():
        q = q_ref[...]
        qn_ref[...] = (
            q / (jnp.sqrt(jnp.sum(q * q, axis=1, keepdims=True)) + 1e-12)
        ).astype(jnp.bfloat16)
        rmin_ref[...] = jnp.full_like(rmin_ref, jnp.inf)
        rjdx_ref[...] = jnp.zeros_like(rjdx_ref)

    k = k_ref[...]
    kn = (k / (jnp.sqrt(jnp.sum(k * k, axis=1, keepdims=True)) + 1e-12)
          ).astype(jnp.bfloat16)

    s = lax.dot_general(
        qn_ref[...], kn,
        dimension_numbers=(((1,), (1,)), ((), ())),
        preferred_element_type=jnp.float32,
    )
    lab = lab_ref[...].reshape(1, KB)
    pen = jnp.where(lab > 0, jnp.float32(1.0), jnp.float32(jnp.inf))
    dd = pen - s

    upd = dd < rmin_ref[...]
    rjdx_ref[...] = jnp.where(upd, j, rjdx_ref[...])
    rmin_ref[...] = jnp.minimum(rmin_ref[...], dd)

    @pl.when(j == nblk - 1)
    def _final():
        rmin = rmin_ref[...]
        m = jnp.min(rmin, axis=1, keepdims=True)
        c = lax.broadcasted_iota(jnp.int32, (Q, KB), 1)
        cand = jnp.where(rmin == m, rjdx_ref[...] * KB + c,
                         jnp.int32(2**31 - 1))
        idx_ref[...] = jnp.min(cand, axis=1, keepdims=True)


def _nearest_index(queries, keys, label_confi):
    K = keys.shape[0]
    nblk = K // KB
    lab3d = label_confi.reshape(K // KB, 1, KB).astype(jnp.int32)
    idx = pl.pallas_call(
        _argmin_body,
        grid=(nblk,),
        in_specs=[
            pl.BlockSpec((Q, D), lambda j: (0, 0)),
            pl.BlockSpec((KB, D), lambda j: (j, 0)),
            pl.BlockSpec((1, 1, KB), lambda j: (j, 0, 0)),
        ],
        out_specs=pl.BlockSpec((Q, 1), lambda j: (0, 0)),
        out_shape=jax.ShapeDtypeStruct((Q, 1), jnp.int32),
        scratch_shapes=[
            pltpu.VMEM((Q, D), jnp.bfloat16),
            pltpu.VMEM((Q, KB), jnp.float32),
            pltpu.VMEM((Q, KB), jnp.int32),
        ],
    )(queries, keys, lab3d)
    return idx.reshape(Q)


def _make_sc_gather(V, B, Dm):
    NC, NS = 2, 16
    NW = NC * NS
    b_per_w = B // NW
    mesh = plsc.VectorSubcoreMesh(core_axis_name="c", subcore_axis_name="s")

    @functools.partial(
        pl.kernel,
        mesh=mesh,
        out_type=jax.ShapeDtypeStruct((B, Dm), jnp.float32),
        scratch_types=[
            pltpu.VMEM((b_per_w,), jnp.int32),
            pltpu.VMEM((b_per_w, Dm), jnp.float32),
            pltpu.SemaphoreType.DMA,
        ],
    )
    def gather_rows(idx_hbm, table_hbm, out_hbm, idx_v, rows_v, sem):
        wid = lax.axis_index("s") * NC + lax.axis_index("c")
        base = wid * b_per_w
        pltpu.sync_copy(idx_hbm.at[pl.ds(base, b_per_w)], idx_v)
        pltpu.async_copy(table_hbm.at[idx_v], rows_v, sem).wait()
        pltpu.sync_copy(rows_v, out_hbm.at[pl.ds(base, b_per_w)])

    return gather_rows


def kernel(queries, keys, label_confi):
    nearest_idx = _nearest_index(queries, keys, label_confi)
    gather = _make_sc_gather(keys.shape[0], Q, D)
    return gather(nearest_idx, keys)
